# Initial kernel scaffold; baseline (speedup 1.0000x reference)
import jax
import jax.numpy as jnp
from jax.experimental import pallas as pl


def kernel(xyz, features):
    raise NotImplementedError("write your pallas kernel here")


# speedup vs baseline: 1.6487x; 1.6487x over previous
import numpy as np

import jax
import jax.numpy as jnp
from jax import lax
from jax.experimental import pallas as pl
from jax.experimental.pallas import tpu as pltpu
from jax.experimental.pallas import tpu_sc as plsc

_INFO = plsc.get_sparse_core_info()
_NC, _NS = _INFO.num_cores, _INFO.num_subcores
_NW = _NC * _NS

_SUB = 128
_SUBS = 7
_CH = _SUB * _SUBS
_NCHW = 14
_PERW = _CH * _NCHW
_ROWS_PAD = _PERW * _NW

_XSUB = 64
_XCHW = 26
_XROWS_PAD = _XSUB * _XCHW * _NW

_MOCK_CONSTS = False


def _gather_body(feat_hbm, xyzt_hbm, gidx_hbm, sidx_hbm, feat_out, xyz_out,
                 idx_v, idxx_v, feat_v, xyz_v, semf, semx):
    wid = lax.axis_index("s") * _NC + lax.axis_index("c")
    for c in range(_NCHW):
        base = (wid * _NCHW + c) * _CH
        for j in range(_SUBS):
            pltpu.sync_copy(gidx_hbm.at[pl.ds(base + j * _SUB, _SUB)],
                            idx_v.at[j])
        cps = [pltpu.async_copy(feat_hbm.at[idx_v.at[j]],
                                feat_v.at[pl.ds(j * _SUB, _SUB)], semf)
               for j in range(_SUBS)]
        for cp in cps:
            cp.wait()
        pltpu.sync_copy(feat_v, feat_out.at[pl.ds(base, _CH)])
    for c in range(_XCHW):
        base = (wid * _XCHW + c) * _XSUB
        pltpu.sync_copy(sidx_hbm.at[pl.ds(base, _XSUB)], idxx_v)
        pltpu.async_copy(xyzt_hbm.at[idxx_v], xyz_v, semx).wait()
        pltpu.sync_copy(xyz_v, xyz_out.at[pl.ds(base, _XSUB)])


def kernel(xyz, features):
    batch, n, _ = xyz.shape
    d = features.shape[-1]
    s = max(1, int(n * 0.5))
    rows = batch * s

    if _MOCK_CONSTS:
        rng = np.random.default_rng(0)
        sidx_np = rng.permutation(n)[:s].astype(np.int32)
        gidx_np = (np.arange(batch, dtype=np.int32)[:, None] * n
                   + sidx_np[None, :]).reshape(rows)
        gidx = jnp.asarray(np.concatenate(
            [gidx_np, np.zeros((_ROWS_PAD - rows,), np.int32)]))
        sidxp = jnp.asarray(np.concatenate(
            [sidx_np, np.zeros((_XROWS_PAD - s,), np.int32)]))
        sidx_b = jnp.asarray(np.tile(sidx_np[None, :], (batch, 1)))
    else:
        with jax.ensure_compile_time_eval():
            perm = jax.random.permutation(jax.random.key(42), n)
            sidx = perm[:s].astype(jnp.int32)
            gidx = (jnp.arange(batch, dtype=jnp.int32)[:, None] * n
                    + sidx[None, :]).reshape(rows)
            gidx = jnp.concatenate(
                [gidx, jnp.zeros((_ROWS_PAD - rows,), jnp.int32)])
            sidxp = jnp.concatenate(
                [sidx, jnp.zeros((_XROWS_PAD - s,), jnp.int32)])
            sidx_b = jnp.tile(sidx[None, :], (batch, 1))

    feat2 = features.reshape(batch * n, d)
    xyzt = jnp.pad(jnp.transpose(xyz, (1, 0, 2)).reshape(n, batch * 3),
                   ((0, 0), (0, 128 - batch * 3)))

    mesh = plsc.VectorSubcoreMesh(core_axis_name="c", subcore_axis_name="s")
    feat_g, xyz_g = pl.kernel(
        _gather_body,
        out_type=[
            jax.ShapeDtypeStruct((_ROWS_PAD, d), jnp.float32),
            jax.ShapeDtypeStruct((_XROWS_PAD, 128), jnp.float32),
        ],
        mesh=mesh,
        scratch_types=[
            pltpu.VMEM((_SUBS, _SUB), jnp.int32),
            pltpu.VMEM((_XSUB,), jnp.int32),
            pltpu.VMEM((_CH, d), jnp.float32),
            pltpu.VMEM((_XSUB, 128), jnp.float32),
            pltpu.SemaphoreType.DMA,
            pltpu.SemaphoreType.DMA,
        ],
    )(feat2, xyzt, gidx, sidxp)

    new_xyz = jnp.transpose(
        xyz_g[:s, :batch * 3].reshape(s, batch, 3), (1, 0, 2))
    return (new_xyz,
            feat_g[:rows].reshape(batch, s, d),
            sidx_b)


# speedup vs baseline: 1.8864x; 1.1442x over previous
import numpy as np

import jax
import jax.numpy as jnp
from jax import lax
from jax.experimental import pallas as pl
from jax.experimental.pallas import tpu as pltpu
from jax.experimental.pallas import tpu_sc as plsc

_INFO = plsc.get_sparse_core_info()
_NC, _NS = _INFO.num_cores, _INFO.num_subcores
_NW = _NC * _NS

_SUB = 128
_SUBS = 2
_CH = _SUB * _SUBS
_NCHW = 49
_PERW = _CH * _NCHW
_ROWS_PAD = _PERW * _NW
_FIDX_ROWS = _NCHW * _SUBS
_FIDX_PAD = 104

_XSUB = 128
_XCHW = 13
_XROWS_PAD = _XSUB * _XCHW * _NW
_XIDX_PAD = 16


def _gather_body(feat_hbm, xyzt_hbm, gidx_hbm, sidx_hbm, feat_out, xyz_out,
                 idxf_v, idxx_v, fb0, fb1, xb0, xb1,
                 semg0, semg1, semw0, semw1):
    wid = lax.axis_index("s") * _NC + lax.axis_index("c")
    pltpu.sync_copy(gidx_hbm.at[wid], idxf_v)
    pltpu.sync_copy(sidx_hbm.at[wid], idxx_v)

    def pipeline(nch, rows, subs, idx_v, bufs, src_hbm, dst_hbm, semg, semw):
        gath = [None, None]
        wb = [None, None]
        for c in range(nch):
            b = c & 1
            if wb[b] is not None:
                wb[b].wait()
                wb[b] = None
            gath[b] = [
                pltpu.async_copy(src_hbm.at[idx_v.at[c * subs + j]],
                                 bufs[b].at[pl.ds(j * _SUB, _SUB)], semg[b])
                for j in range(subs)]
            if c >= 1:
                p = 1 - b
                for g in gath[p]:
                    g.wait()
                gath[p] = None
                wb[p] = pltpu.async_copy(
                    bufs[p],
                    dst_hbm.at[pl.ds((wid * nch + c - 1) * rows, rows)],
                    semw[p])
        last = (nch - 1) & 1
        for g in gath[last]:
            g.wait()
        pltpu.sync_copy(bufs[last],
                        dst_hbm.at[pl.ds((wid * nch + nch - 1) * rows, rows)])
        for b in (0, 1):
            if wb[b] is not None:
                wb[b].wait()

    pipeline(_NCHW, _CH, _SUBS, idxf_v, (fb0, fb1), feat_hbm, feat_out,
             (semg0, semg1), (semw0, semw1))
    pipeline(_XCHW, _XSUB, 1, idxx_v, (xb0, xb1), xyzt_hbm, xyz_out,
             (semg0, semg1), (semw0, semw1))


def kernel(xyz, features):
    batch, n, _ = xyz.shape
    d = features.shape[-1]
    s = max(1, int(n * 0.5))
    rows = batch * s

    with jax.ensure_compile_time_eval():
        perm = jax.random.permutation(jax.random.key(42), n)
        sidx = perm[:s].astype(jnp.int32)
        gidx = (jnp.arange(batch, dtype=jnp.int32)[:, None] * n
                + sidx[None, :]).reshape(rows)
        gidx = jnp.concatenate(
            [gidx, jnp.zeros((_ROWS_PAD - rows,), jnp.int32)])
        gidx = jnp.pad(gidx.reshape(_NW, _FIDX_ROWS, _SUB),
                       ((0, 0), (0, _FIDX_PAD - _FIDX_ROWS), (0, 0)))
        sidxp = jnp.concatenate(
            [sidx, jnp.zeros((_XROWS_PAD - s,), jnp.int32)])
        sidxp = jnp.pad(sidxp.reshape(_NW, _XCHW, _SUB),
                        ((0, 0), (0, _XIDX_PAD - _XCHW), (0, 0)))
        sidx_b = jnp.tile(sidx[None, :], (batch, 1))

    feat2 = features.reshape(batch * n, d)
    xyzt = jnp.pad(jnp.transpose(xyz, (1, 0, 2)).reshape(n, batch * 3),
                   ((0, 0), (0, 128 - batch * 3)))

    mesh = plsc.VectorSubcoreMesh(core_axis_name="c", subcore_axis_name="s")
    feat_g, xyz_g = pl.kernel(
        _gather_body,
        out_type=[
            jax.ShapeDtypeStruct((_ROWS_PAD, d), jnp.float32),
            jax.ShapeDtypeStruct((_XROWS_PAD, 128), jnp.float32),
        ],
        mesh=mesh,
        scratch_types=[
            pltpu.VMEM((_FIDX_PAD, _SUB), jnp.int32),
            pltpu.VMEM((_XIDX_PAD, _SUB), jnp.int32),
            pltpu.VMEM((_CH, d), jnp.float32),
            pltpu.VMEM((_CH, d), jnp.float32),
            pltpu.VMEM((_XSUB, 128), jnp.float32),
            pltpu.VMEM((_XSUB, 128), jnp.float32),
            pltpu.SemaphoreType.DMA,
            pltpu.SemaphoreType.DMA,
            pltpu.SemaphoreType.DMA,
            pltpu.SemaphoreType.DMA,
        ],
    )(feat2, xyzt, gidx, sidxp)

    new_xyz = jnp.transpose(
        xyz_g[:s, :batch * 3].reshape(s, batch, 3), (1, 0, 2))
    return (new_xyz,
            feat_g[:rows].reshape(batch, s, d),
            sidx_b)


# speedup vs baseline: 3.7108x; 1.9671x over previous
import numpy as np

import jax
import jax.numpy as jnp
from jax import lax
from jax.experimental import pallas as pl
from jax.experimental.pallas import tpu as pltpu
from jax.experimental.pallas import tpu_sc as plsc

_INFO = plsc.get_sparse_core_info()
_NC, _NS = _INFO.num_cores, _INFO.num_subcores
_NW = _NC * _NS

_SUB = 128
_FCH = 3125
_FFULL = _FCH // _NW
_FREM = _FCH % _NW
_FIDX_PAD = 104

_XFULL_ROWS = 50000 // _SUB
_XTAIL = 50000 - _XFULL_ROWS * _SUB
_XFULL = _XFULL_ROWS // _NW
_XREM = _XFULL_ROWS % _NW
_XIDX_PAD = 16


def _gather_body(feat_hbm, xyzt_hbm, gidx_hbm, sidx_hbm, feat_out, xyz_out,
                 idxf_v, idxx_v, fb0, fb1, xb0, xb1,
                 semg0, semg1, semw0, semw1):
    wid = lax.axis_index("s") * _NC + lax.axis_index("c")
    pltpu.sync_copy(gidx_hbm.at[wid], idxf_v)
    pltpu.sync_copy(sidx_hbm.at[wid], idxx_v)

    def pipeline(nfull, idx_v, bufs, src_hbm, dst_hbm, semg, semw):
        gath = [None, None]
        wb = [None, None]
        for c in range(nfull):
            b = c & 1
            if wb[b] is not None:
                wb[b].wait()
                wb[b] = None
            gath[b] = pltpu.async_copy(
                src_hbm.at[idx_v.at[c]], bufs[b], semg[b])
            if c >= 1:
                p = 1 - b
                gath[p].wait()
                gath[p] = None
                wb[p] = pltpu.async_copy(
                    bufs[p],
                    dst_hbm.at[pl.ds(((c - 1) * _NW + wid) * _SUB, _SUB)],
                    semw[p])
        last = (nfull - 1) & 1
        gath[last].wait()
        pltpu.sync_copy(
            bufs[last],
            dst_hbm.at[pl.ds(((nfull - 1) * _NW + wid) * _SUB, _SUB)])
        for b in (0, 1):
            if wb[b] is not None:
                wb[b].wait()

    pipeline(_FFULL, idxf_v, (fb0, fb1), feat_hbm, feat_out,
             (semg0, semg1), (semw0, semw1))

    @pl.when(wid < _FREM)
    def _():
        pltpu.async_copy(feat_hbm.at[idxf_v.at[_FFULL]], fb0, semg0).wait()
        pltpu.sync_copy(
            fb0, feat_out.at[pl.ds((_FFULL * _NW + wid) * _SUB, _SUB)])

    pipeline(_XFULL, idxx_v, (xb0, xb1), xyzt_hbm, xyz_out,
             (semg0, semg1), (semw0, semw1))

    @pl.when(wid < _XREM)
    def _():
        pltpu.async_copy(xyzt_hbm.at[idxx_v.at[_XFULL]], xb0, semg0).wait()
        pltpu.sync_copy(
            xb0, xyz_out.at[pl.ds((_XFULL * _NW + wid) * _SUB, _SUB)])

    @pl.when(wid == _XREM)
    def _():
        pltpu.async_copy(xyzt_hbm.at[idxx_v.at[_XFULL]], xb1, semg1).wait()
        pltpu.sync_copy(xb1.at[pl.ds(0, _XTAIL)],
                        xyz_out.at[pl.ds(_XFULL_ROWS * _SUB, _XTAIL)])


def kernel(xyz, features):
    batch, n, _ = xyz.shape
    d = features.shape[-1]
    s = max(1, int(n * 0.5))
    rows = batch * s

    with jax.ensure_compile_time_eval():
        perm = jax.random.permutation(jax.random.key(42), n)
        sidx = perm[:s].astype(jnp.int32)
        sidx_b = jnp.tile(sidx[None, :], (batch, 1))

        gidx = (jnp.arange(batch, dtype=jnp.int32)[:, None] * n
                + sidx[None, :]).reshape(_FCH, _SUB)
        gidx = jnp.pad(gidx, ((0, _NW * _FIDX_PAD - _FCH), (0, 0)))
        gidx = gidx.reshape(_FIDX_PAD, _NW, _SUB).transpose(1, 0, 2)

        sidxp = jnp.pad(sidx, (0, _NW * _XIDX_PAD * _SUB - s))
        sidxp = sidxp.reshape(_XIDX_PAD, _NW, _SUB).transpose(1, 0, 2)

    feat2 = features.reshape(batch * n, d)
    xyzt = jnp.pad(jnp.transpose(xyz, (1, 0, 2)).reshape(n, batch * 3),
                   ((0, 0), (0, 128 - batch * 3)))

    mesh = plsc.VectorSubcoreMesh(core_axis_name="c", subcore_axis_name="s")
    feat_g, xyz_g = pl.kernel(
        _gather_body,
        out_type=[
            jax.ShapeDtypeStruct((rows, d), jnp.float32),
            jax.ShapeDtypeStruct((s, 128), jnp.float32),
        ],
        mesh=mesh,
        scratch_types=[
            pltpu.VMEM((_FIDX_PAD, _SUB), jnp.int32),
            pltpu.VMEM((_XIDX_PAD, _SUB), jnp.int32),
            pltpu.VMEM((_SUB, d), jnp.float32),
            pltpu.VMEM((_SUB, d), jnp.float32),
            pltpu.VMEM((_SUB, 128), jnp.float32),
            pltpu.VMEM((_SUB, 128), jnp.float32),
            pltpu.SemaphoreType.DMA,
            pltpu.SemaphoreType.DMA,
            pltpu.SemaphoreType.DMA,
            pltpu.SemaphoreType.DMA,
        ],
    )(feat2, xyzt, gidx, sidxp)

    new_xyz = jnp.transpose(
        xyz_g[:, :batch * 3].reshape(s, batch, 3), (1, 0, 2))
    return (new_xyz,
            feat_g.reshape(batch, s, d),
            sidx_b)


# speedup vs baseline: 3.7213x; 1.0028x over previous
import numpy as np

import jax
import jax.numpy as jnp
from jax import lax
from jax.experimental import pallas as pl
from jax.experimental.pallas import tpu as pltpu
from jax.experimental.pallas import tpu_sc as plsc

_INFO = plsc.get_sparse_core_info()
_NC, _NS = _INFO.num_cores, _INFO.num_subcores
_NW = _NC * _NS

_SUB = 128
_FCH = 3125
_FFULL = _FCH // _NW
_FREM = _FCH % _NW
_FIDX_PAD = 104

_XFULL_ROWS = 50000 // _SUB
_XTAIL = 50000 - _XFULL_ROWS * _SUB
_XFULL = _XFULL_ROWS // _NW
_XREM = _XFULL_ROWS % _NW
_XIDX_PAD = 16


def _gather_body(feat_hbm, xyzt_hbm, gidx_hbm, sidx_hbm, feat_out, xyz_out,
                 idxf_v, idxx_v, fb0, fb1, fb2, xb0, xb1,
                 semg0, semg1, semg2, semw0, semw1, semw2):
    wid = lax.axis_index("s") * _NC + lax.axis_index("c")
    pltpu.sync_copy(gidx_hbm.at[wid], idxf_v)
    pltpu.sync_copy(sidx_hbm.at[wid], idxx_v)

    def pipeline(nfull, idx_v, bufs, src_hbm, dst_hbm, semg, semw):
        depth = len(bufs)
        gath = [None] * depth
        wb = [None] * depth

        def retire(pc):
            p = pc % depth
            gath[p].wait()
            gath[p] = None
            wb[p] = pltpu.async_copy(
                bufs[p],
                dst_hbm.at[pl.ds((pc * _NW + wid) * _SUB, _SUB)],
                semw[p])

        for c in range(nfull):
            b = c % depth
            if wb[b] is not None:
                wb[b].wait()
                wb[b] = None
            gath[b] = pltpu.async_copy(
                src_hbm.at[idx_v.at[c]], bufs[b], semg[b])
            if c - (depth - 1) >= 0:
                retire(c - (depth - 1))
        for pc in range(max(0, nfull - (depth - 1)), nfull):
            retire(pc)
        for b in range(depth):
            if wb[b] is not None:
                wb[b].wait()

    pipeline(_FFULL, idxf_v, (fb0, fb1, fb2), feat_hbm, feat_out,
             (semg0, semg1, semg2), (semw0, semw1, semw2))

    @pl.when(wid < _FREM)
    def _():
        pltpu.async_copy(feat_hbm.at[idxf_v.at[_FFULL]], fb0, semg0).wait()
        pltpu.sync_copy(
            fb0, feat_out.at[pl.ds((_FFULL * _NW + wid) * _SUB, _SUB)])

    pipeline(_XFULL, idxx_v, (xb0, xb1), xyzt_hbm, xyz_out,
             (semg0, semg1), (semw0, semw1))

    @pl.when(wid < _XREM)
    def _():
        pltpu.async_copy(xyzt_hbm.at[idxx_v.at[_XFULL]], xb0, semg0).wait()
        pltpu.sync_copy(
            xb0, xyz_out.at[pl.ds((_XFULL * _NW + wid) * _SUB, _SUB)])

    @pl.when(wid == _XREM)
    def _():
        pltpu.async_copy(xyzt_hbm.at[idxx_v.at[_XFULL]], xb1, semg1).wait()
        pltpu.sync_copy(xb1.at[pl.ds(0, _XTAIL)],
                        xyz_out.at[pl.ds(_XFULL_ROWS * _SUB, _XTAIL)])


def kernel(xyz, features):
    batch, n, _ = xyz.shape
    d = features.shape[-1]
    s = max(1, int(n * 0.5))
    rows = batch * s

    with jax.ensure_compile_time_eval():
        perm = jax.random.permutation(jax.random.key(42), n)
        sidx = perm[:s].astype(jnp.int32)
        sidx_b = jnp.tile(sidx[None, :], (batch, 1))

        gidx = (jnp.arange(batch, dtype=jnp.int32)[:, None] * n
                + sidx[None, :]).reshape(_FCH, _SUB)
        gidx = jnp.pad(gidx, ((0, _NW * _FIDX_PAD - _FCH), (0, 0)))
        gidx = gidx.reshape(_FIDX_PAD, _NW, _SUB).transpose(1, 0, 2)

        sidxp = jnp.pad(sidx, (0, _NW * _XIDX_PAD * _SUB - s))
        sidxp = sidxp.reshape(_XIDX_PAD, _NW, _SUB).transpose(1, 0, 2)

    feat2 = features.reshape(batch * n, d)
    xyzt = jnp.pad(jnp.transpose(xyz, (1, 0, 2)).reshape(n, batch * 3),
                   ((0, 0), (0, 128 - batch * 3)))

    mesh = plsc.VectorSubcoreMesh(core_axis_name="c", subcore_axis_name="s")
    feat_g, xyz_g = pl.kernel(
        _gather_body,
        out_type=[
            jax.ShapeDtypeStruct((rows, d), jnp.float32),
            jax.ShapeDtypeStruct((s, 128), jnp.float32),
        ],
        mesh=mesh,
        scratch_types=[
            pltpu.VMEM((_FIDX_PAD, _SUB), jnp.int32),
            pltpu.VMEM((_XIDX_PAD, _SUB), jnp.int32),
            pltpu.VMEM((_SUB, d), jnp.float32),
            pltpu.VMEM((_SUB, d), jnp.float32),
            pltpu.VMEM((_SUB, d), jnp.float32),
            pltpu.VMEM((_SUB, 128), jnp.float32),
            pltpu.VMEM((_SUB, 128), jnp.float32),
            pltpu.SemaphoreType.DMA,
            pltpu.SemaphoreType.DMA,
            pltpu.SemaphoreType.DMA,
            pltpu.SemaphoreType.DMA,
            pltpu.SemaphoreType.DMA,
            pltpu.SemaphoreType.DMA,
        ],
    )(feat2, xyzt, gidx, sidxp)

    new_xyz = jnp.transpose(
        xyz_g[:, :batch * 3].reshape(s, batch, 3), (1, 0, 2))
    return (new_xyz,
            feat_g.reshape(batch, s, d),
            sidx_b)
